# Initial kernel scaffold; baseline (speedup 1.0000x reference)
import jax
import jax.numpy as jnp
from jax.experimental import pallas as pl


def kernel(output, target):
    raise NotImplementedError("write your pallas kernel here")


# speedup vs baseline: 2.1280x; 2.1280x over previous
import functools

import jax
import jax.numpy as jnp
from jax.experimental import pallas as pl
from jax.experimental.pallas import tpu as pltpu

SCALE = 30.0
MARGIN = 0.35


def _lmcl_body(C, W, B, x_ref, tgt_ref, o_ref, m_scr, s_scr, xt_scr):
    j = pl.program_id(0)
    nj = pl.num_programs(0)

    @pl.when(j == 0)
    def _init():
        m_scr[...] = jnp.full((B, 1), -jnp.inf, jnp.float32)
        s_scr[...] = jnp.zeros((B, 1), jnp.float32)
        xt_scr[...] = jnp.zeros((B, 1), jnp.float32)

    yb = x_ref[...] * SCALE
    col_ids = j * W + jax.lax.broadcasted_iota(jnp.int32, (B, W), 1)
    valid = col_ids < C
    yb = jnp.where(valid, yb, -jnp.inf)

    eq = col_ids == tgt_ref[...]
    xt_scr[...] += jnp.sum(jnp.where(eq, yb, 0.0), axis=1, keepdims=True)

    m_old = m_scr[...]
    m_new = jnp.maximum(m_old, jnp.max(yb, axis=1, keepdims=True))
    p = jnp.exp(yb - m_new)
    s_new = s_scr[...] * jnp.exp(m_old - m_new) + jnp.sum(
        p, axis=1, keepdims=True
    )
    m_scr[...] = m_new
    s_scr[...] = s_new

    @pl.when(j == nj - 1)
    def _fin():
        yt = xt_scr[...]
        ytm = yt - SCALE * MARGIN
        m = m_scr[...]
        s_corr = s_scr[...] - jnp.exp(yt - m) + jnp.exp(ytm - m)
        nll = m + jnp.log(s_corr) - ytm
        o_ref[...] = jnp.sum(nll, axis=0, keepdims=True) / B


def kernel(output, target):
    B, C = output.shape
    W = 1024
    nj = pl.cdiv(C, W)
    tgt = target.astype(jnp.int32).reshape(B, 1)

    out = pl.pallas_call(
        functools.partial(_lmcl_body, C, W, B),
        grid=(nj,),
        in_specs=[
            pl.BlockSpec((B, W), lambda j: (0, j)),
            pl.BlockSpec((B, 1), lambda j: (0, 0)),
        ],
        out_specs=pl.BlockSpec((1, 1), lambda j: (0, 0)),
        out_shape=jax.ShapeDtypeStruct((1, 1), jnp.float32),
        scratch_shapes=[
            pltpu.VMEM((B, 1), jnp.float32),
            pltpu.VMEM((B, 1), jnp.float32),
            pltpu.VMEM((B, 1), jnp.float32),
        ],
    )(output, tgt)
    return out[0, 0]
